# baseline (speedup 1.0000x reference)
import functools

import jax
import jax.numpy as jnp
from jax import lax
from jax.experimental import pallas as pl
from jax.experimental.pallas import tpu as pltpu
from jax.experimental.pallas import tpu_sc as plsc

B = 4096
F = 26
D = 64
FD = F * D
H1, H2, H3 = 1024, 512, 256

NC = 2
NS = 16
NW = NC * NS
TOTAL_ROWS = B * F
ROWS_PER_W = TOTAL_ROWS // NW
CH = 128
CHUNKS = ROWS_PER_W // CH


def _sc_gather(idx3d, table):
    mesh = plsc.VectorSubcoreMesh(core_axis_name="c", subcore_axis_name="s")

    @functools.partial(
        pl.kernel,
        mesh=mesh,
        out_type=jax.ShapeDtypeStruct((TOTAL_ROWS, D), jnp.float32),
        scratch_types=[
            pltpu.VMEM((CHUNKS, CH), jnp.int32),
            pltpu.VMEM((CH, D), jnp.float32),
            pltpu.SemaphoreType.DMA,
        ],
        compiler_params=pltpu.CompilerParams(use_tc_tiling_on_sc=False),
    )
    def gather_kernel(idx_hbm, table_hbm, out_hbm, idx_v, rows_v, sem):
        w = lax.axis_index("s") * NC + lax.axis_index("c")
        pltpu.sync_copy(idx_hbm.at[w], idx_v)
        row0 = w * ROWS_PER_W

        def body(j, carry):
            pltpu.async_copy(table_hbm.at[idx_v.at[j]], rows_v, sem).wait()
            pltpu.sync_copy(rows_v, out_hbm.at[pl.ds(row0 + j * CH, CH)])
            return carry

        lax.fori_loop(0, CHUNKS, body, 0)

    return gather_kernel(idx3d, table)


def _mlp_body(e_ref, a_ref, w1_ref, b1_ref, w2_ref, b2_ref, w3_ref, b3_ref,
              wo_ref, bo_ref, out_ref):
    e = e_ref[...]
    s = jnp.dot(e, a_ref[...], preferred_element_type=jnp.float32)
    fm = 0.5 * (jnp.sum(s * s, axis=1) - jnp.sum(e * e, axis=1))
    h = e.astype(jnp.bfloat16)
    h = jnp.dot(h, w1_ref[...], preferred_element_type=jnp.float32) + b1_ref[...]
    h = jnp.maximum(h, 0.0).astype(jnp.bfloat16)
    h = jnp.dot(h, w2_ref[...], preferred_element_type=jnp.float32) + b2_ref[...]
    h = jnp.maximum(h, 0.0).astype(jnp.bfloat16)
    h = jnp.dot(h, w3_ref[...], preferred_element_type=jnp.float32) + b3_ref[...]
    h = jnp.maximum(h, 0.0)
    mlp = jnp.sum(h * wo_ref[...], axis=1) + bo_ref[0, 0]
    logit = fm + mlp
    out_ref[...] = 1.0 / (1.0 + jnp.exp(-logit))


def _tc_mlp(emb, a, w1, b1, w2, b2, w3, b3, wo_row, bo):
    bm = 512
    grid = (B // bm,)
    const = lambda i: (0, 0)
    return pl.pallas_call(
        _mlp_body,
        grid=grid,
        in_specs=[
            pl.BlockSpec((bm, FD), lambda i: (i, 0)),
            pl.BlockSpec((FD, D), const),
            pl.BlockSpec((FD, H1), const),
            pl.BlockSpec((1, H1), const),
            pl.BlockSpec((H1, H2), const),
            pl.BlockSpec((1, H2), const),
            pl.BlockSpec((H2, H3), const),
            pl.BlockSpec((1, H3), const),
            pl.BlockSpec((1, H3), const),
            pl.BlockSpec((1, 1), const),
        ],
        out_specs=pl.BlockSpec((bm,), lambda i: (i,)),
        out_shape=jax.ShapeDtypeStruct((B,), jnp.float32),
    )(emb, a, w1, b1, w2, b2, w3, b3, wo_row, bo)


def kernel(x, table, W1, b1, W2, b2, W3, b3, Wo, bo):
    idx3d = x.astype(jnp.int32).reshape(NW, CHUNKS, CH)
    emb_flat = _sc_gather(idx3d, table)
    emb = emb_flat.reshape(B, FD)
    a = jnp.tile(jnp.eye(D, dtype=jnp.float32), (F, 1))
    return _tc_mlp(
        emb, a,
        W1.astype(jnp.bfloat16), b1.reshape(1, H1),
        W2.astype(jnp.bfloat16), b2.reshape(1, H2),
        W3.astype(jnp.bfloat16), b3.reshape(1, H3),
        Wo.reshape(1, H3).astype(jnp.float32), bo.reshape(1, 1),
    )
